# baseline (device time: 172959 ns/iter reference)
import functools

import jax
import jax.numpy as jnp
from jax import lax
from jax.experimental import pallas as pl
from jax.experimental.pallas import tpu as pltpu

N_DEV = 16


def kernel(A, B):
    m, k = A.shape
    _, n = B.shape
    rows = m // N_DEV

    def body(a_ref, b_ref, out_ref, a_bf, b_bf, recvbuf, sendbuf,
             send_sems, recv_sems):
        my = lax.axis_index("i")
        left = (my - 1) % N_DEV
        right = (my + 1) % N_DEV

        barrier = pltpu.get_barrier_semaphore()
        for nbr in (left, right):
            pl.semaphore_signal(barrier, inc=1, device_id=(nbr,),
                                device_id_type=pl.DeviceIdType.MESH)
        pl.semaphore_wait(barrier, 2)

        a_bf[...] = a_ref[...].astype(jnp.bfloat16)
        b_bf[...] = b_ref[...].astype(jnp.bfloat16)

        def part(c):
            return jnp.dot(a_bf[pl.ds(c * rows, rows), :], b_bf[...],
                           preferred_element_type=jnp.float32)

        acc = part(my)
        for s in range(N_DEV - 1):
            sendbuf[...] = acc.astype(jnp.bfloat16)
            rdma = pltpu.make_async_remote_copy(
                src_ref=sendbuf,
                dst_ref=recvbuf.at[s],
                send_sem=send_sems.at[s],
                recv_sem=recv_sems.at[s],
                device_id=(right,),
                device_id_type=pl.DeviceIdType.MESH,
            )
            rdma.start()
            nxt = part((my - s - 1) % N_DEV)
            rdma.wait()
            acc = nxt + recvbuf[s].astype(jnp.float32)

        owned = (my + 1) % N_DEV
        z = acc
        g = 0.5 * z * (1.0 + jnp.tanh(0.7978845608 * (z + 0.044715 * z * z * z)))
        out_ref[pl.ds(owned * rows, rows), :] = g
        sendbuf[...] = g.astype(jnp.bfloat16)

        for h in range(N_DEV - 1):
            slot = (N_DEV - 1) + h
            src = sendbuf if h == 0 else recvbuf.at[slot - 1]
            rdma = pltpu.make_async_remote_copy(
                src_ref=src,
                dst_ref=recvbuf.at[slot],
                send_sem=send_sems.at[slot],
                recv_sem=recv_sems.at[slot],
                device_id=(right,),
                device_id_type=pl.DeviceIdType.MESH,
            )
            rdma.start()
            rdma.wait()
            cidx = (my - h) % N_DEV
            out_ref[pl.ds(cidx * rows, rows), :] = recvbuf[slot].astype(jnp.float32)

        @functools.partial(pl.run_scoped, exit_sem=pltpu.SemaphoreType.REGULAR)
        def _(exit_sem):
            for nbr in (left, right):
                pl.semaphore_signal(exit_sem, inc=1, device_id=(nbr,),
                                    device_id_type=pl.DeviceIdType.MESH)
            pl.semaphore_wait(exit_sem, 2)

    n_slots = 2 * (N_DEV - 1)
    return pl.pallas_call(
        body,
        out_shape=jax.ShapeDtypeStruct((m, n), jnp.float32),
        in_specs=[pl.BlockSpec(memory_space=pltpu.VMEM),
                  pl.BlockSpec(memory_space=pltpu.VMEM)],
        out_specs=pl.BlockSpec(memory_space=pltpu.VMEM),
        scratch_shapes=[
            pltpu.VMEM((m, k), jnp.bfloat16),
            pltpu.VMEM((k, n), jnp.bfloat16),
            pltpu.VMEM((n_slots, rows, n), jnp.bfloat16),
            pltpu.VMEM((rows, n), jnp.bfloat16),
            pltpu.SemaphoreType.DMA((n_slots,)),
            pltpu.SemaphoreType.DMA((n_slots,)),
        ],
        compiler_params=pltpu.CompilerParams(collective_id=0),
    )(A, B)


# device time: 98309 ns/iter; 1.7593x vs baseline; 1.7593x over previous
import jax
import jax.numpy as jnp
from jax import lax
from jax.experimental import pallas as pl
from jax.experimental.pallas import tpu as pltpu

N_DEV = 16
PLANE = 4
NZ = 4
QROWS = 384
CROWS = 96
HCOLS = 768


def kernel(A, B):
    m, k = A.shape
    _, n = B.shape

    def body(a_ref, b_ref, out_ref, a_bf, b_bf,
             prs_recv, prs_stage, zrs_recv, zrs_stage,
             zag_recv, q0, pag_recv, pacc,
             prs_ssem, prs_rsem, zrs_ssem, zrs_rsem,
             zag_ssem, zag_rsem, pag_ssem, pag_rsem):
        my = lax.axis_index("i")
        z = my // PLANE
        j = my % PLANE
        pn = PLANE * z + (j + 1) % PLANE
        pp = PLANE * z + (j - 1) % PLANE
        zn = PLANE * ((z + 1) % NZ) + j
        zp = PLANE * ((z - 1) % NZ) + j

        barrier = pltpu.get_barrier_semaphore()
        for nbr in (pn, pp, zn, zp):
            pl.semaphore_signal(barrier, inc=1, device_id=(nbr,),
                                device_id_type=pl.DeviceIdType.MESH)
        pl.semaphore_wait(barrier, 4)

        a_bf[...] = a_ref[...].astype(jnp.bfloat16)
        b_bf[...] = b_ref[...].astype(jnp.bfloat16)

        def qpart(q, d):
            bcols = b_bf[:, d * HCOLS:(d + 1) * HCOLS]
            return jnp.dot(a_bf[pl.ds(q * QROWS, QROWS), :], bcols,
                           preferred_element_type=jnp.float32)

        def rdma(src, dst, ssem, rsem, dev):
            return pltpu.make_async_remote_copy(
                src_ref=src, dst_ref=dst, send_sem=ssem, recv_sem=rsem,
                device_id=(dev,), device_id_type=pl.DeviceIdType.MESH)

        acc0 = qpart(j, 0)
        acc1 = qpart(j, 1)
        for s in range(PLANE - 1):
            prs_stage[0] = acc0.astype(jnp.bfloat16)
            prs_stage[1] = acc1.astype(jnp.bfloat16)
            r0 = rdma(prs_stage.at[0], prs_recv.at[0, s],
                      prs_ssem.at[0, s], prs_rsem.at[0, s], pn)
            r1 = rdma(prs_stage.at[1], prs_recv.at[1, s],
                      prs_ssem.at[1, s], prs_rsem.at[1, s], pp)
            r0.start()
            r1.start()
            l0 = qpart((j - s - 1) % PLANE, 0)
            l1 = qpart((j + s + 1) % PLANE, 1)
            r0.wait()
            r1.wait()
            acc0 = l0 + prs_recv[0, s].astype(jnp.float32)
            acc1 = l1 + prs_recv[1, s].astype(jnp.float32)

        pacc[0] = acc0
        pacc[1] = acc1

        def zch(d, c):
            return pacc[d, pl.ds(c * CROWS, CROWS), :]

        zacc0 = zch(0, z)
        zacc1 = zch(1, z)
        for s in range(NZ - 1):
            zrs_stage[0] = zacc0.astype(jnp.bfloat16)
            zrs_stage[1] = zacc1.astype(jnp.bfloat16)
            r0 = rdma(zrs_stage.at[0], zrs_recv.at[0, s],
                      zrs_ssem.at[0, s], zrs_rsem.at[0, s], zn)
            r1 = rdma(zrs_stage.at[1], zrs_recv.at[1, s],
                      zrs_ssem.at[1, s], zrs_rsem.at[1, s], zp)
            r0.start()
            r1.start()
            l0 = zch(0, (z - s - 1) % NZ)
            l1 = zch(1, (z + s + 1) % NZ)
            r0.wait()
            r1.wait()
            zacc0 = l0 + zrs_recv[0, s].astype(jnp.float32)
            zacc1 = l1 + zrs_recv[1, s].astype(jnp.float32)

        def gelu(v):
            return 0.5 * v * (1.0 + jnp.tanh(
                0.7978845608 * (v + 0.044715 * v * v * v)))

        g0 = gelu(zacc0)
        g1 = gelu(zacc1)
        q_own0 = (j + 1) % PLANE
        q_own1 = (j - 1) % PLANE
        c_own0 = (z + 1) % NZ
        c_own1 = (z - 1) % NZ
        out_ref[pl.ds(q_own0 * QROWS + c_own0 * CROWS, CROWS), 0:HCOLS] = g0
        out_ref[pl.ds(q_own1 * QROWS + c_own1 * CROWS, CROWS), HCOLS:n] = g1
        q0[0, pl.ds(c_own0 * CROWS, CROWS), :] = g0.astype(jnp.bfloat16)
        q0[1, pl.ds(c_own1 * CROWS, CROWS), :] = g1.astype(jnp.bfloat16)
        zrs_stage[0] = g0.astype(jnp.bfloat16)
        zrs_stage[1] = g1.astype(jnp.bfloat16)

        for h in range(NZ - 1):
            src0 = zrs_stage.at[0] if h == 0 else zag_recv.at[0, h - 1]
            src1 = zrs_stage.at[1] if h == 0 else zag_recv.at[1, h - 1]
            r0 = rdma(src0, zag_recv.at[0, h],
                      zag_ssem.at[0, h], zag_rsem.at[0, h], zn)
            r1 = rdma(src1, zag_recv.at[1, h],
                      zag_ssem.at[1, h], zag_rsem.at[1, h], zp)
            r0.start()
            r1.start()
            r0.wait()
            r1.wait()
            c0 = (z - h) % NZ
            c1 = (z + h) % NZ
            out_ref[pl.ds(q_own0 * QROWS + c0 * CROWS, CROWS), 0:HCOLS] = (
                zag_recv[0, h].astype(jnp.float32))
            out_ref[pl.ds(q_own1 * QROWS + c1 * CROWS, CROWS), HCOLS:n] = (
                zag_recv[1, h].astype(jnp.float32))
            q0[0, pl.ds(c0 * CROWS, CROWS), :] = zag_recv[0, h]
            q0[1, pl.ds(c1 * CROWS, CROWS), :] = zag_recv[1, h]

        for h in range(PLANE - 1):
            src0 = q0.at[0] if h == 0 else pag_recv.at[0, h - 1]
            src1 = q0.at[1] if h == 0 else pag_recv.at[1, h - 1]
            r0 = rdma(src0, pag_recv.at[0, h],
                      pag_ssem.at[0, h], pag_rsem.at[0, h], pn)
            r1 = rdma(src1, pag_recv.at[1, h],
                      pag_ssem.at[1, h], pag_rsem.at[1, h], pp)
            r0.start()
            r1.start()
            r0.wait()
            r1.wait()
            q_0 = (j - h) % PLANE
            q_1 = (j + h) % PLANE
            out_ref[pl.ds(q_0 * QROWS, QROWS), 0:HCOLS] = (
                pag_recv[0, h].astype(jnp.float32))
            out_ref[pl.ds(q_1 * QROWS, QROWS), HCOLS:n] = (
                pag_recv[1, h].astype(jnp.float32))

    return pl.pallas_call(
        body,
        out_shape=jax.ShapeDtypeStruct((m, n), jnp.float32),
        in_specs=[pl.BlockSpec(memory_space=pltpu.VMEM),
                  pl.BlockSpec(memory_space=pltpu.VMEM)],
        out_specs=pl.BlockSpec(memory_space=pltpu.VMEM),
        scratch_shapes=[
            pltpu.VMEM((m, k), jnp.bfloat16),
            pltpu.VMEM((k, n), jnp.bfloat16),
            pltpu.VMEM((2, 3, QROWS, HCOLS), jnp.bfloat16),
            pltpu.VMEM((2, QROWS, HCOLS), jnp.bfloat16),
            pltpu.VMEM((2, 3, CROWS, HCOLS), jnp.bfloat16),
            pltpu.VMEM((2, CROWS, HCOLS), jnp.bfloat16),
            pltpu.VMEM((2, 3, CROWS, HCOLS), jnp.bfloat16),
            pltpu.VMEM((2, QROWS, HCOLS), jnp.bfloat16),
            pltpu.VMEM((2, 3, QROWS, HCOLS), jnp.bfloat16),
            pltpu.VMEM((2, QROWS, HCOLS), jnp.float32),
            pltpu.SemaphoreType.DMA((2, 3)),
            pltpu.SemaphoreType.DMA((2, 3)),
            pltpu.SemaphoreType.DMA((2, 3)),
            pltpu.SemaphoreType.DMA((2, 3)),
            pltpu.SemaphoreType.DMA((2, 3)),
            pltpu.SemaphoreType.DMA((2, 3)),
            pltpu.SemaphoreType.DMA((2, 3)),
            pltpu.SemaphoreType.DMA((2, 3)),
        ],
        compiler_params=pltpu.CompilerParams(collective_id=0),
    )(A, B)


# device time: 90279 ns/iter; 1.9158x vs baseline; 1.0889x over previous
import jax
import jax.numpy as jnp
from jax import lax
from jax.experimental import pallas as pl
from jax.experimental.pallas import tpu as pltpu

N_DEV = 16
PLANE = 4
NZ = 4
QROWS = 384
CROWS = 96
HCOLS = 768


def kernel(A, B):
    m, k = A.shape
    _, n = B.shape

    def body(a_ref, b_ref, out_ref, a_bf, b_bf,
             prs_recv, prs_stage, zrs_recv, zrs_stage,
             zag_recv, pag_recv, pacc,
             prs_ssem, prs_rsem, zrs_ssem, zrs_rsem,
             zag_ssem, zag_rsem, pag_ssem, pag_rsem):
        my = lax.axis_index("i")
        z = my // PLANE
        j = my % PLANE
        pn = PLANE * z + (j + 1) % PLANE
        pp = PLANE * z + (j - 1) % PLANE
        zn = PLANE * ((z + 1) % NZ) + j
        zp = PLANE * ((z - 1) % NZ) + j

        barrier = pltpu.get_barrier_semaphore()
        for nbr in (pn, pp, zn, zp):
            pl.semaphore_signal(barrier, inc=1, device_id=(nbr,),
                                device_id_type=pl.DeviceIdType.MESH)
        pl.semaphore_wait(barrier, 4)

        a_bf[...] = a_ref[...].astype(jnp.bfloat16)
        b_bf[...] = b_ref[...].astype(jnp.bfloat16)

        def qpart(q, d):
            bcols = b_bf[:, d * HCOLS:(d + 1) * HCOLS]
            return jnp.dot(a_bf[pl.ds(q * QROWS, QROWS), :], bcols,
                           preferred_element_type=jnp.float32)

        def rdma(src, dst, ssem, rsem, dev):
            return pltpu.make_async_remote_copy(
                src_ref=src, dst_ref=dst, send_sem=ssem, recv_sem=rsem,
                device_id=(dev,), device_id_type=pl.DeviceIdType.MESH)

        acc0 = qpart(j, 0)
        acc1 = qpart(j, 1)
        for s in range(PLANE - 1):
            prs_stage[0] = acc0.astype(jnp.bfloat16)
            prs_stage[1] = acc1.astype(jnp.bfloat16)
            r0 = rdma(prs_stage.at[0], prs_recv.at[0, s],
                      prs_ssem.at[0, s], prs_rsem.at[0, s], pn)
            r1 = rdma(prs_stage.at[1], prs_recv.at[1, s],
                      prs_ssem.at[1, s], prs_rsem.at[1, s], pp)
            r0.start()
            r1.start()
            l0 = qpart((j - s - 1) % PLANE, 0)
            l1 = qpart((j + s + 1) % PLANE, 1)
            r0.wait()
            r1.wait()
            acc0 = l0 + prs_recv[0, s].astype(jnp.float32)
            acc1 = l1 + prs_recv[1, s].astype(jnp.float32)

        pacc[0] = acc0
        pacc[1] = acc1

        def zch(d, c):
            return pacc[d, pl.ds(c * CROWS, CROWS), :]

        zacc0 = zch(0, z)
        zacc1 = zch(1, z)
        for s in range(NZ - 1):
            zrs_stage[0] = zacc0.astype(jnp.bfloat16)
            zrs_stage[1] = zacc1.astype(jnp.bfloat16)
            r0 = rdma(zrs_stage.at[0], zrs_recv.at[0, s],
                      zrs_ssem.at[0, s], zrs_rsem.at[0, s], zn)
            r1 = rdma(zrs_stage.at[1], zrs_recv.at[1, s],
                      zrs_ssem.at[1, s], zrs_rsem.at[1, s], zp)
            r0.start()
            r1.start()
            l0 = zch(0, (z - s - 1) % NZ)
            l1 = zch(1, (z + s + 1) % NZ)
            r0.wait()
            r1.wait()
            zacc0 = l0 + zrs_recv[0, s].astype(jnp.float32)
            zacc1 = l1 + zrs_recv[1, s].astype(jnp.float32)

        def gelu(v):
            return 0.5 * v * (1.0 + jnp.tanh(
                0.7978845608 * (v + 0.044715 * v * v * v)))

        g0 = gelu(zacc0)
        g1 = gelu(zacc1)
        q_own0 = (j + 1) % PLANE
        q_own1 = (j - 1) % PLANE
        c_own0 = (z + 1) % NZ
        c_own1 = (z - 1) % NZ
        out_ref[pl.ds(q_own0 * QROWS + c_own0 * CROWS, CROWS), 0:HCOLS] = g0
        out_ref[pl.ds(q_own1 * QROWS + c_own1 * CROWS, CROWS), HCOLS:n] = g1
        zrs_stage[0] = g0.astype(jnp.bfloat16)
        zrs_stage[1] = g1.astype(jnp.bfloat16)

        def zmsg(d, ztgt, h):
            src = zrs_stage.at[d] if h == 0 else zag_recv.at[d, h - 1]
            return rdma(src, zag_recv.at[d, h],
                        zag_ssem.at[d, h], zag_rsem.at[d, h], ztgt)

        def pmsg(d, ptgt, c, h):
            if h == 0:
                src = zrs_stage.at[d] if c == 0 else zag_recv.at[d, c - 1]
            else:
                src = pag_recv.at[d, c, h - 1]
            return rdma(src, pag_recv.at[d, c, h],
                        pag_ssem.at[d, c, h], pag_rsem.at[d, c, h], ptgt)

        def zrow(d, c):
            return ((z + 1 - c) if d == 0 else (z - 1 + c)) % NZ

        zdirs = ((0, zn), (1, zp))
        pdirs = ((0, pn), (1, pp))
        for R in range(NZ + PLANE - 2):
            msgs = []
            if R <= NZ - 2:
                for d, ztgt in zdirs:
                    msgs.append(("z", d, R, zmsg(d, ztgt, R)))
            for d, ptgt in pdirs:
                for c in range(min(R, NZ - 1) + 1):
                    h = R - c
                    if h <= PLANE - 2:
                        msgs.append(("p", d, (c, h), pmsg(d, ptgt, c, h)))
            for _, _, _, r in msgs:
                r.start()
            for kind, d, key, r in msgs:
                r.wait()
                lo, hi = (0, HCOLS) if d == 0 else (HCOLS, n)
                if kind == "z":
                    h = key
                    q_idx = q_own0 if d == 0 else q_own1
                    row = q_idx * QROWS + zrow(d, h + 1) * CROWS
                    out_ref[pl.ds(row, CROWS), lo:hi] = (
                        zag_recv[d, h].astype(jnp.float32))
                else:
                    c, h = key
                    q_idx = ((j - h) if d == 0 else (j + h)) % PLANE
                    row = q_idx * QROWS + zrow(d, c) * CROWS
                    out_ref[pl.ds(row, CROWS), lo:hi] = (
                        pag_recv[d, c, h].astype(jnp.float32))

    return pl.pallas_call(
        body,
        out_shape=jax.ShapeDtypeStruct((m, n), jnp.float32),
        in_specs=[pl.BlockSpec(memory_space=pltpu.VMEM),
                  pl.BlockSpec(memory_space=pltpu.VMEM)],
        out_specs=pl.BlockSpec(memory_space=pltpu.VMEM),
        scratch_shapes=[
            pltpu.VMEM((m, k), jnp.bfloat16),
            pltpu.VMEM((k, n), jnp.bfloat16),
            pltpu.VMEM((2, 3, QROWS, HCOLS), jnp.bfloat16),
            pltpu.VMEM((2, QROWS, HCOLS), jnp.bfloat16),
            pltpu.VMEM((2, 3, CROWS, HCOLS), jnp.bfloat16),
            pltpu.VMEM((2, CROWS, HCOLS), jnp.bfloat16),
            pltpu.VMEM((2, 3, CROWS, HCOLS), jnp.bfloat16),
            pltpu.VMEM((2, 4, 3, CROWS, HCOLS), jnp.bfloat16),
            pltpu.VMEM((2, QROWS, HCOLS), jnp.float32),
            pltpu.SemaphoreType.DMA((2, 3)),
            pltpu.SemaphoreType.DMA((2, 3)),
            pltpu.SemaphoreType.DMA((2, 3)),
            pltpu.SemaphoreType.DMA((2, 3)),
            pltpu.SemaphoreType.DMA((2, 3)),
            pltpu.SemaphoreType.DMA((2, 3)),
            pltpu.SemaphoreType.DMA((2, 4, 3)),
            pltpu.SemaphoreType.DMA((2, 4, 3)),
        ],
        compiler_params=pltpu.CompilerParams(collective_id=0),
    )(A, B)


# device time: 84822 ns/iter; 2.0391x vs baseline; 1.0643x over previous
import jax
import jax.numpy as jnp
from jax import lax
from jax.experimental import pallas as pl
from jax.experimental.pallas import tpu as pltpu

N_DEV = 16
PLANE = 4
NZ = 4
QROWS = 384
CROWS = 96
HCOLS = 768


def kernel(A, B):
    m, k = A.shape
    _, n = B.shape

    def body(a_ref, b_ref, out_ref, a_bf, b_bf,
             prs_send, prs_recv, zrs_send, zrs_recv, pred, ag_stage,
             zag_recv, pag_recv,
             prs_ssem, prs_rsem, zrs_ssem, zrs_rsem,
             zag_ssem, zag_rsem, pag_ssem, pag_rsem):
        my = lax.axis_index("i")
        z = my // PLANE
        j = my % PLANE
        pn = PLANE * z + (j + 1) % PLANE
        pp = PLANE * z + (j - 1) % PLANE
        zn = PLANE * ((z + 1) % NZ) + j
        zp = PLANE * ((z - 1) % NZ) + j

        barrier = pltpu.get_barrier_semaphore()
        for nbr in (pn, pp, zn, zp):
            pl.semaphore_signal(barrier, inc=1, device_id=(nbr,),
                                device_id_type=pl.DeviceIdType.MESH)
        pl.semaphore_wait(barrier, 4)

        a_bf[...] = a_ref[...].astype(jnp.bfloat16)
        b_bf[...] = b_ref[...].astype(jnp.bfloat16)

        def rdma(src, dst, ssem, rsem, dev):
            return pltpu.make_async_remote_copy(
                src_ref=src, dst_ref=dst, send_sem=ssem, recv_sem=rsem,
                device_id=(dev,), device_id_type=pl.DeviceIdType.MESH)

        def ch(d, u):
            return ((z - u) if d == 0 else (z + u)) % NZ

        def cpart(d, qq, u):
            bcols = b_bf[:, d * HCOLS:(d + 1) * HCOLS]
            arows = a_bf[pl.ds(qq * QROWS + ch(d, u) * CROWS, CROWS), :]
            return jnp.dot(arows, bcols, preferred_element_type=jnp.float32)

        def own_q(d):
            return j

        def recv_q(d, s):
            return ((j - s - 1) if d == 0 else (j + s + 1)) % PLANE

        pdirs = ((0, pn), (1, pp))
        zdirs = ((0, zn), (1, zp))

        for d, _ in pdirs:
            prs_send[d, 0, 0] = cpart(d, own_q(d), 0).astype(jnp.bfloat16)

        zacc = [None, None]
        for R in range(6):
            msgs = []
            for d, ptgt in pdirs:
                for u in range(NZ):
                    s = R - u
                    if 0 <= s <= PLANE - 2:
                        msgs.append(("p", d, (s, u), rdma(
                            prs_send.at[d, s, u], prs_recv.at[d, s, u],
                            prs_ssem.at[d, s, u], prs_rsem.at[d, s, u],
                            ptgt)))
            t = R - 3
            if 0 <= t <= NZ - 2:
                for d, ztgt in zdirs:
                    msgs.append(("z", d, t, rdma(
                        zrs_send.at[d, t], zrs_recv.at[d, t],
                        zrs_ssem.at[d, t], zrs_rsem.at[d, t], ztgt)))
            for _, _, _, r in msgs:
                r.start()
            if R + 1 <= NZ - 1:
                for d, _ in pdirs:
                    prs_send[d, 0, R + 1] = (
                        cpart(d, own_q(d), R + 1).astype(jnp.bfloat16))
            locs = {}
            for kind, d, key, _ in msgs:
                if kind == "p":
                    s, u = key
                    locs[(d, s, u)] = cpart(d, recv_q(d, s), u)
            for kind, d, key, r in msgs:
                r.wait()
                if kind == "p":
                    s, u = key
                    val = locs[(d, s, u)] + prs_recv[d, s, u].astype(jnp.float32)
                    if s < PLANE - 2:
                        prs_send[d, s + 1, u] = val.astype(jnp.bfloat16)
                    else:
                        pred[d, u] = val
                        if u == 0:
                            zrs_send[d, 0] = val.astype(jnp.bfloat16)
                else:
                    t = key
                    if t < NZ - 2:
                        nxt = (pred[d, t + 1]
                               + zrs_recv[d, t].astype(jnp.float32))
                        zrs_send[d, t + 1] = nxt.astype(jnp.bfloat16)
                    else:
                        zacc[d] = (pred[d, NZ - 1]
                                   + zrs_recv[d, t].astype(jnp.float32))

        def gelu(v):
            return 0.5 * v * (1.0 + jnp.tanh(
                0.7978845608 * (v + 0.044715 * v * v * v)))

        g0 = gelu(zacc[0])
        g1 = gelu(zacc[1])
        q_own0 = (j + 1) % PLANE
        q_own1 = (j - 1) % PLANE
        out_ref[pl.ds(q_own0 * QROWS + ((z + 1) % NZ) * CROWS, CROWS),
                0:HCOLS] = g0
        out_ref[pl.ds(q_own1 * QROWS + ((z - 1) % NZ) * CROWS, CROWS),
                HCOLS:n] = g1
        ag_stage[0] = g0.astype(jnp.bfloat16)
        ag_stage[1] = g1.astype(jnp.bfloat16)

        def zmsg(d, ztgt, h):
            src = ag_stage.at[d] if h == 0 else zag_recv.at[d, h - 1]
            return rdma(src, zag_recv.at[d, h],
                        zag_ssem.at[d, h], zag_rsem.at[d, h], ztgt)

        def pmsg(d, ptgt, c, h):
            if h == 0:
                src = ag_stage.at[d] if c == 0 else zag_recv.at[d, c - 1]
            else:
                src = pag_recv.at[d, c, h - 1]
            return rdma(src, pag_recv.at[d, c, h],
                        pag_ssem.at[d, c, h], pag_rsem.at[d, c, h], ptgt)

        def zrow(d, c):
            return ((z + 1 - c) if d == 0 else (z - 1 + c)) % NZ

        for R in range(NZ + PLANE - 2):
            msgs = []
            if R <= NZ - 2:
                for d, ztgt in zdirs:
                    msgs.append(("z", d, R, zmsg(d, ztgt, R)))
            for d, ptgt in pdirs:
                for c in range(min(R, NZ - 1) + 1):
                    h = R - c
                    if h <= PLANE - 2:
                        msgs.append(("p", d, (c, h), pmsg(d, ptgt, c, h)))
            for _, _, _, r in msgs:
                r.start()
            for kind, d, key, r in msgs:
                r.wait()
                lo, hi = (0, HCOLS) if d == 0 else (HCOLS, n)
                if kind == "z":
                    h = key
                    q_idx = q_own0 if d == 0 else q_own1
                    row = q_idx * QROWS + zrow(d, h + 1) * CROWS
                    out_ref[pl.ds(row, CROWS), lo:hi] = (
                        zag_recv[d, h].astype(jnp.float32))
                else:
                    c, h = key
                    q_idx = ((j - h) if d == 0 else (j + h)) % PLANE
                    row = q_idx * QROWS + zrow(d, c) * CROWS
                    out_ref[pl.ds(row, CROWS), lo:hi] = (
                        pag_recv[d, c, h].astype(jnp.float32))

    return pl.pallas_call(
        body,
        out_shape=jax.ShapeDtypeStruct((m, n), jnp.float32),
        in_specs=[pl.BlockSpec(memory_space=pltpu.VMEM),
                  pl.BlockSpec(memory_space=pltpu.VMEM)],
        out_specs=pl.BlockSpec(memory_space=pltpu.VMEM),
        scratch_shapes=[
            pltpu.VMEM((m, k), jnp.bfloat16),
            pltpu.VMEM((k, n), jnp.bfloat16),
            pltpu.VMEM((2, 3, 4, CROWS, HCOLS), jnp.bfloat16),
            pltpu.VMEM((2, 3, 4, CROWS, HCOLS), jnp.bfloat16),
            pltpu.VMEM((2, 3, CROWS, HCOLS), jnp.bfloat16),
            pltpu.VMEM((2, 3, CROWS, HCOLS), jnp.bfloat16),
            pltpu.VMEM((2, 4, CROWS, HCOLS), jnp.float32),
            pltpu.VMEM((2, CROWS, HCOLS), jnp.bfloat16),
            pltpu.VMEM((2, 3, CROWS, HCOLS), jnp.bfloat16),
            pltpu.VMEM((2, 4, 3, CROWS, HCOLS), jnp.bfloat16),
            pltpu.SemaphoreType.DMA((2, 3, 4)),
            pltpu.SemaphoreType.DMA((2, 3, 4)),
            pltpu.SemaphoreType.DMA((2, 3)),
            pltpu.SemaphoreType.DMA((2, 3)),
            pltpu.SemaphoreType.DMA((2, 3)),
            pltpu.SemaphoreType.DMA((2, 3)),
            pltpu.SemaphoreType.DMA((2, 4, 3)),
            pltpu.SemaphoreType.DMA((2, 4, 3)),
        ],
        compiler_params=pltpu.CompilerParams(collective_id=0),
    )(A, B)


# device time: 82655 ns/iter; 2.0925x vs baseline; 1.0262x over previous
import jax
import jax.numpy as jnp
from jax import lax
from jax.experimental import pallas as pl
from jax.experimental.pallas import tpu as pltpu

N_DEV = 16
PLANE = 4
NZ = 4
QROWS = 384
CROWS = 96
HCOLS = 768


def kernel(A, B):
    m, k = A.shape
    _, n = B.shape

    def body(a_ref, b_ref, out_ref, a_bf, b_bf,
             prs_send, prs_recv, zrs_send, zrs_recv, pred, ag_stage,
             zag_recv, pag_recv,
             prs_ssem, prs_rsem, zrs_ssem, zrs_rsem,
             zag_ssem, zag_rsem, pag_ssem, pag_rsem):
        my = lax.axis_index("i")
        z = my // PLANE
        j = my % PLANE
        pn = PLANE * z + (j + 1) % PLANE
        pp = PLANE * z + (j - 1) % PLANE
        zn = PLANE * ((z + 1) % NZ) + j
        zp = PLANE * ((z - 1) % NZ) + j

        barrier = pltpu.get_barrier_semaphore()
        for nbr in (pn, pp, zn, zp):
            pl.semaphore_signal(barrier, inc=1, device_id=(nbr,),
                                device_id_type=pl.DeviceIdType.MESH)
        pl.semaphore_wait(barrier, 4)

        a_bf[...] = a_ref[...].astype(jnp.bfloat16)
        b_bf[...] = b_ref[...].astype(jnp.bfloat16)

        def rdma(src, dst, ssem, rsem, dev):
            return pltpu.make_async_remote_copy(
                src_ref=src, dst_ref=dst, send_sem=ssem, recv_sem=rsem,
                device_id=(dev,), device_id_type=pl.DeviceIdType.MESH)

        def ch(d, u):
            return ((z - u) if d == 0 else (z + u)) % NZ

        def cpart(d, qq, u):
            bcols = b_bf[:, d * HCOLS:(d + 1) * HCOLS]
            arows = a_bf[pl.ds(qq * QROWS + ch(d, u) * CROWS, CROWS), :]
            return jnp.dot(arows, bcols, preferred_element_type=jnp.float32)

        def own_q(d):
            return j

        def recv_q(d, s):
            return ((j - s - 1) if d == 0 else (j + s + 1)) % PLANE

        pdirs = ((0, pn), (1, pp))
        zdirs = ((0, zn), (1, zp))

        for u in range(NZ):
            for d, ptgt in pdirs:
                prs_send[d, 0, u] = cpart(d, own_q(d), u).astype(jnp.bfloat16)
                rdma(prs_send.at[d, 0, u], prs_recv.at[d, 0, u],
                     prs_ssem.at[d, 0, u], prs_rsem.at[d, 0, u],
                     ptgt).start()

        zacc = [None, None]
        for R in range(6):
            msgs = []
            for d, ptgt in pdirs:
                for u in range(NZ):
                    s = R - u
                    if 0 <= s <= PLANE - 2:
                        r = rdma(
                            prs_send.at[d, s, u], prs_recv.at[d, s, u],
                            prs_ssem.at[d, s, u], prs_rsem.at[d, s, u],
                            ptgt)
                        msgs.append(("p", d, (s, u), r, s > 0))
            t = R - 3
            if 0 <= t <= NZ - 2:
                for d, ztgt in zdirs:
                    msgs.append(("z", d, t, rdma(
                        zrs_send.at[d, t], zrs_recv.at[d, t],
                        zrs_ssem.at[d, t], zrs_rsem.at[d, t], ztgt), True))
            for _, _, _, r, needs_start in msgs:
                if needs_start:
                    r.start()
            locs = {}
            for kind, d, key, _, _ in msgs:
                if kind == "p":
                    s, u = key
                    locs[(d, s, u)] = cpart(d, recv_q(d, s), u)
            for kind, d, key, r, _ in msgs:
                r.wait()
                if kind == "p":
                    s, u = key
                    val = locs[(d, s, u)] + prs_recv[d, s, u].astype(jnp.float32)
                    if s < PLANE - 2:
                        prs_send[d, s + 1, u] = val.astype(jnp.bfloat16)
                    else:
                        pred[d, u] = val
                        if u == 0:
                            zrs_send[d, 0] = val.astype(jnp.bfloat16)
                else:
                    t = key
                    if t < NZ - 2:
                        nxt = (pred[d, t + 1]
                               + zrs_recv[d, t].astype(jnp.float32))
                        zrs_send[d, t + 1] = nxt.astype(jnp.bfloat16)
                    else:
                        zacc[d] = (pred[d, NZ - 1]
                                   + zrs_recv[d, t].astype(jnp.float32))

        def gelu(v):
            return 0.5 * v * (1.0 + jnp.tanh(
                0.7978845608 * (v + 0.044715 * v * v * v)))

        g0 = gelu(zacc[0])
        g1 = gelu(zacc[1])
        q_own0 = (j + 1) % PLANE
        q_own1 = (j - 1) % PLANE
        out_ref[pl.ds(q_own0 * QROWS + ((z + 1) % NZ) * CROWS, CROWS),
                0:HCOLS] = g0
        out_ref[pl.ds(q_own1 * QROWS + ((z - 1) % NZ) * CROWS, CROWS),
                HCOLS:n] = g1
        ag_stage[0] = g0.astype(jnp.bfloat16)
        ag_stage[1] = g1.astype(jnp.bfloat16)

        def zmsg(d, ztgt, h):
            src = ag_stage.at[d] if h == 0 else zag_recv.at[d, h - 1]
            return rdma(src, zag_recv.at[d, h],
                        zag_ssem.at[d, h], zag_rsem.at[d, h], ztgt)

        def pmsg(d, ptgt, c, h):
            if h == 0:
                src = ag_stage.at[d] if c == 0 else zag_recv.at[d, c - 1]
            else:
                src = pag_recv.at[d, c, h - 1]
            return rdma(src, pag_recv.at[d, c, h],
                        pag_ssem.at[d, c, h], pag_rsem.at[d, c, h], ptgt)

        def zrow(d, c):
            return ((z + 1 - c) if d == 0 else (z - 1 + c)) % NZ

        def flush(writes):
            for kind, d, key in writes:
                lo, hi = (0, HCOLS) if d == 0 else (HCOLS, n)
                if kind == "z":
                    h = key
                    q_idx = q_own0 if d == 0 else q_own1
                    row = q_idx * QROWS + zrow(d, h + 1) * CROWS
                    out_ref[pl.ds(row, CROWS), lo:hi] = (
                        zag_recv[d, h].astype(jnp.float32))
                else:
                    c, h = key
                    q_idx = ((j - h) if d == 0 else (j + h)) % PLANE
                    row = q_idx * QROWS + zrow(d, c) * CROWS
                    out_ref[pl.ds(row, CROWS), lo:hi] = (
                        pag_recv[d, c, h].astype(jnp.float32))

        pending = []
        for R in range(NZ + PLANE - 2):
            msgs = []
            if R <= NZ - 2:
                for d, ztgt in zdirs:
                    msgs.append(("z", d, R, zmsg(d, ztgt, R)))
            for d, ptgt in pdirs:
                for c in range(min(R, NZ - 1) + 1):
                    h = R - c
                    if h <= PLANE - 2:
                        msgs.append(("p", d, (c, h), pmsg(d, ptgt, c, h)))
            for _, _, _, r in msgs:
                r.start()
            flush(pending)
            pending = []
            for kind, d, key, r in msgs:
                r.wait()
                pending.append((kind, d, key))
        flush(pending)

    return pl.pallas_call(
        body,
        out_shape=jax.ShapeDtypeStruct((m, n), jnp.float32),
        in_specs=[pl.BlockSpec(memory_space=pltpu.VMEM),
                  pl.BlockSpec(memory_space=pltpu.VMEM)],
        out_specs=pl.BlockSpec(memory_space=pltpu.VMEM),
        scratch_shapes=[
            pltpu.VMEM((m, k), jnp.bfloat16),
            pltpu.VMEM((k, n), jnp.bfloat16),
            pltpu.VMEM((2, 3, 4, CROWS, HCOLS), jnp.bfloat16),
            pltpu.VMEM((2, 3, 4, CROWS, HCOLS), jnp.bfloat16),
            pltpu.VMEM((2, 3, CROWS, HCOLS), jnp.bfloat16),
            pltpu.VMEM((2, 3, CROWS, HCOLS), jnp.bfloat16),
            pltpu.VMEM((2, 4, CROWS, HCOLS), jnp.float32),
            pltpu.VMEM((2, CROWS, HCOLS), jnp.bfloat16),
            pltpu.VMEM((2, 3, CROWS, HCOLS), jnp.bfloat16),
            pltpu.VMEM((2, 4, 3, CROWS, HCOLS), jnp.bfloat16),
            pltpu.SemaphoreType.DMA((2, 3, 4)),
            pltpu.SemaphoreType.DMA((2, 3, 4)),
            pltpu.SemaphoreType.DMA((2, 3)),
            pltpu.SemaphoreType.DMA((2, 3)),
            pltpu.SemaphoreType.DMA((2, 3)),
            pltpu.SemaphoreType.DMA((2, 3)),
            pltpu.SemaphoreType.DMA((2, 4, 3)),
            pltpu.SemaphoreType.DMA((2, 4, 3)),
        ],
        compiler_params=pltpu.CompilerParams(collective_id=0),
    )(A, B)
